# baseline (speedup 1.0000x reference)
import jax
import jax.numpy as jnp
from jax import lax
from jax.experimental import pallas as pl
from jax.experimental.pallas import tpu as pltpu
from jax.experimental.pallas import tpu_sc as plsc

_N = 10000
_D = 128
_H = _D // 2
_E = 320000
_LANES = 16

_NC = 2
_NS = 16
_EDGES_PER_CORE = _E // _NC
_EDGES_PER_TILE = _E // (_NC * _NS)
_K = 80
_CHUNKS = _EDGES_PER_TILE // _K
_ROWS_PER_TILE = 632
_NPAD = _ROWS_PER_TILE * _NS


def _spmm_body(ha_hbm, hb_hbm, src_hbm, dst_hbm, w_hbm, out_hbm,
               src_all, dst_all, w_all, rows0, rows1, zbuf, accum,
               sem0, sem1, ssem0, ssem1):
    cid = lax.axis_index("c")
    sid = lax.axis_index("s")
    row0 = sid * _ROWS_PER_TILE
    chunk0 = cid * (_EDGES_PER_CORE // _K) + sid * _CHUNKS

    pltpu.sync_copy(src_hbm.at[pl.ds(chunk0, _CHUNKS)], src_all)
    pltpu.sync_copy(dst_hbm.at[pl.ds(chunk0, _CHUNKS)], dst_all)
    pltpu.sync_copy(w_hbm.at[pl.ds(chunk0, _CHUNKS)], w_all)

    def _zrow(r, carry):
        for j in range(_H // _LANES):
            zbuf[r, pl.ds(j * _LANES, _LANES)] = jnp.zeros((_LANES,), jnp.float32)
        return carry
    lax.fori_loop(0, _ROWS_PER_TILE, _zrow, 0)

    for half, h_hbm in enumerate((ha_hbm, hb_hbm)):
        pltpu.sync_copy(zbuf, accum.at[pl.ds(row0, _ROWS_PER_TILE)])
        plsc.subcore_barrier()

        def _gather(c, buf, sem):
            pltpu.async_copy(h_hbm.at[src_all.at[c]], buf, sem)

        def _wait_g(c, buf, sem):
            pltpu.make_async_copy(h_hbm.at[src_all.at[c]], buf, sem).wait()

        def _scale(c, buf):
            def _grp(g, c2):
                w16 = w_all[c, pl.ds(g * _LANES, _LANES)]
                e0 = g * _LANES
                for i in range(_LANES):
                    wv = jnp.full((_LANES,), w16[i], jnp.float32)
                    for j in range(_H // _LANES):
                        sl = pl.ds(j * _LANES, _LANES)
                        buf[e0 + i, sl] = buf[e0 + i, sl] * wv
                return c2
            lax.fori_loop(0, _K // _LANES, _grp, 0)

        def _scatter(c, buf, sem):
            pltpu.async_copy(buf, accum.at[dst_all.at[c]], sem, add=True)

        def _wait_s(c, buf, sem):
            pltpu.make_async_copy(buf, accum.at[dst_all.at[c]], sem).wait()

        _gather(0, rows0, sem0)
        _gather(1, rows1, sem1)

        def _pair(j, carry):
            c0 = 2 * j
            _wait_g(c0, rows0, sem0)
            _scale(c0, rows0)
            _scatter(c0, rows0, ssem0)
            _wait_g(c0 + 1, rows1, sem1)
            _scale(c0 + 1, rows1)
            _scatter(c0 + 1, rows1, ssem1)
            _wait_s(c0, rows0, ssem0)
            _gather(c0 + 2, rows0, sem0)
            _wait_s(c0 + 1, rows1, ssem1)

            @pl.when(c0 + 3 < _CHUNKS)
            def _():
                _gather(c0 + 3, rows1, sem1)
            return carry
        lax.fori_loop(0, (_CHUNKS - 1) // 2, _pair, 0)
        _wait_g(_CHUNKS - 1, rows0, sem0)
        _scale(_CHUNKS - 1, rows0)
        pltpu.sync_copy(rows0, accum.at[dst_all.at[_CHUNKS - 1]], add=True)

        plsc.subcore_barrier()
        out0 = (cid * 2 + half) * _NPAD + row0
        pltpu.sync_copy(accum.at[pl.ds(row0, _ROWS_PER_TILE)],
                        out_hbm.at[pl.ds(out0, _ROWS_PER_TILE)])
        plsc.subcore_barrier()


_spmm = pl.kernel(
    _spmm_body,
    out_type=jax.ShapeDtypeStruct((_NC * 2 * _NPAD, _H), jnp.float32),
    mesh=plsc.VectorSubcoreMesh(core_axis_name="c", subcore_axis_name="s"),
    compiler_params=pltpu.CompilerParams(use_tc_tiling_on_sc=False),
    scratch_types=[
        pltpu.VMEM((_CHUNKS, _K), jnp.int32),
        pltpu.VMEM((_CHUNKS, _K), jnp.int32),
        pltpu.VMEM((_CHUNKS, _K), jnp.float32),
        pltpu.VMEM((_K, _H), jnp.float32),
        pltpu.VMEM((_K, _H), jnp.float32),
        pltpu.VMEM((_ROWS_PER_TILE, _H), jnp.float32),
        pltpu.VMEM_SHARED((_NPAD, _H), jnp.float32),
        pltpu.SemaphoreType.DMA,
        pltpu.SemaphoreType.DMA,
        pltpu.SemaphoreType.DMA,
        pltpu.SemaphoreType.DMA,
    ],
)


_BN = 1000


def _lin1_body(x_ref, wt_ref, b_ref, o_ref):
    o_ref[...] = (jnp.dot(x_ref[...], wt_ref[...],
                          preferred_element_type=jnp.float32) + b_ref[...])


def _lin2_body(p00_ref, p01_ref, p10_ref, p11_ref, wt_ref, b_ref, o_ref):
    hl = jnp.maximum(p00_ref[...] + p10_ref[...], 0.0)
    hr = jnp.maximum(p01_ref[...] + p11_ref[...], 0.0)
    acc = jnp.dot(hl, wt_ref[:_H, :], preferred_element_type=jnp.float32)
    acc += jnp.dot(hr, wt_ref[_H:, :], preferred_element_type=jnp.float32)
    o_ref[...] = acc + b_ref[...]


def _add_body(a0_ref, a1_ref, b0_ref, b1_ref, o_ref):
    o_ref[:, :_H] = a0_ref[...] + b0_ref[...]
    o_ref[:, _H:] = a1_ref[...] + b1_ref[...]


_row_spec = pl.BlockSpec((_BN, _D), lambda i: (i, 0))
_half_spec = pl.BlockSpec((_BN, _H), lambda i: (i, 0))
_mat_spec = pl.BlockSpec((_D, _D), lambda i: (0, 0))
_bias_spec = pl.BlockSpec((1, _D), lambda i: (0, 0))
_out_f32 = jax.ShapeDtypeStruct((_N, _D), jnp.float32)

_linear1 = pl.pallas_call(
    _lin1_body, grid=(_N // _BN,),
    in_specs=[_row_spec, _mat_spec, _bias_spec],
    out_specs=_row_spec, out_shape=_out_f32)

_linear2 = pl.pallas_call(
    _lin2_body, grid=(_N // _BN,),
    in_specs=[_half_spec, _half_spec, _half_spec, _half_spec,
              _mat_spec, _bias_spec],
    out_specs=_row_spec, out_shape=_out_f32)

_addk = pl.pallas_call(
    _add_body, grid=(_N // _BN,),
    in_specs=[_half_spec, _half_spec, _half_spec, _half_spec],
    out_specs=_row_spec, out_shape=_out_f32)


def _partials(p):
    return [p[k * _NPAD:k * _NPAD + _N] for k in range(_NC * 2)]


def kernel(x, edge_index, edge_weight, W1, b1, W2, b2):
    src = edge_index[0].reshape(_E // _K, _K)
    dst = edge_index[1].reshape(_E // _K, _K)
    edge_weight = edge_weight.reshape(_E // _K, _K)
    w1t = W1.T
    w2t = W2.T
    b1r = b1.reshape(1, _D)
    b2r = b2.reshape(1, _D)

    h1 = _linear1(x, w1t, b1r)
    p = _partials(_spmm(h1[:, :_H], h1[:, _H:], src, dst, edge_weight))
    h2 = _linear2(p[0], p[1], p[2], p[3], w2t, b2r)
    q = _partials(_spmm(h2[:, :_H], h2[:, _H:], src, dst, edge_weight))
    return _addk(q[0], q[1], q[2], q[3])


# speedup vs baseline: 1.7375x; 1.7375x over previous
import jax
import jax.numpy as jnp
from jax import lax
from jax.experimental import pallas as pl
from jax.experimental.pallas import tpu as pltpu
from jax.experimental.pallas import tpu_sc as plsc

_N = 10000
_D = 128
_H = _D // 2
_E = 320000
_LANES = 16

_NC = 2
_NS = 16
_EDGES_PER_CORE = _E // _NC
_EDGES_PER_TILE = _E // (_NC * _NS)
_K = 80
_CHUNKS = _EDGES_PER_TILE // _K
_ROWS_PER_TILE = 632
_NPAD = _ROWS_PER_TILE * _NS


def _spmm_body(ha_hbm, hb_hbm, src_hbm, dst_hbm, w_hbm, out_hbm,
               src_all, dst_all, w_all, rows0, rows1, zbuf, accum,
               sem0, sem1):
    cid = lax.axis_index("c")
    sid = lax.axis_index("s")
    row0 = sid * _ROWS_PER_TILE
    chunk0 = cid * (_EDGES_PER_CORE // _K) + sid * _CHUNKS

    pltpu.sync_copy(src_hbm.at[pl.ds(chunk0, _CHUNKS)], src_all)
    pltpu.sync_copy(dst_hbm.at[pl.ds(chunk0, _CHUNKS)], dst_all)
    pltpu.sync_copy(w_hbm.at[pl.ds(chunk0, _CHUNKS)], w_all)

    def _zrow(r, carry):
        for j in range(_H // _LANES):
            zbuf[r, pl.ds(j * _LANES, _LANES)] = jnp.zeros((_LANES,), jnp.float32)
        return carry
    lax.fori_loop(0, _ROWS_PER_TILE, _zrow, 0)

    for half, h_hbm in enumerate((ha_hbm, hb_hbm)):
        pltpu.sync_copy(zbuf, accum.at[pl.ds(row0, _ROWS_PER_TILE)])
        plsc.subcore_barrier()

        def _gather(c, buf, sem):
            pltpu.async_copy(h_hbm.at[src_all.at[c]], buf, sem)

        def _wait_g(c, buf, sem):
            pltpu.make_async_copy(h_hbm.at[src_all.at[c]], buf, sem).wait()

        def _scale(c, buf):
            def _grp(g, c2):
                w16 = w_all[c, pl.ds(g * _LANES, _LANES)]
                e0 = g * _LANES
                nb = _H // _LANES
                for i in range(0, _LANES, 2):
                    wva = jnp.full((_LANES,), w16[i], jnp.float32)
                    wvb = jnp.full((_LANES,), w16[i + 1], jnp.float32)
                    va = [buf[e0 + i, pl.ds(j * _LANES, _LANES)]
                          for j in range(nb)]
                    vb = [buf[e0 + i + 1, pl.ds(j * _LANES, _LANES)]
                          for j in range(nb)]
                    for j in range(nb):
                        buf[e0 + i, pl.ds(j * _LANES, _LANES)] = va[j] * wva
                    for j in range(nb):
                        buf[e0 + i + 1, pl.ds(j * _LANES, _LANES)] = vb[j] * wvb
                return c2
            lax.fori_loop(0, _K // _LANES, _grp, 0)

        def _process(c, buf):
            _scale(c, buf)
            pltpu.sync_copy(buf, accum.at[dst_all.at[c]], add=True)

        _gather(0, rows0, sem0)

        def _pair(j, carry):
            c0 = 2 * j
            _wait_g(c0, rows0, sem0)
            _gather(c0 + 1, rows1, sem1)
            _process(c0, rows0)
            _wait_g(c0 + 1, rows1, sem1)
            _gather(c0 + 2, rows0, sem0)
            _process(c0 + 1, rows1)
            return carry
        lax.fori_loop(0, (_CHUNKS - 1) // 2, _pair, 0)
        _wait_g(_CHUNKS - 1, rows0, sem0)
        _process(_CHUNKS - 1, rows0)

        plsc.subcore_barrier()
        out0 = (cid * 2 + half) * _NPAD + row0
        pltpu.sync_copy(accum.at[pl.ds(row0, _ROWS_PER_TILE)],
                        out_hbm.at[pl.ds(out0, _ROWS_PER_TILE)])
        plsc.subcore_barrier()


_spmm = pl.kernel(
    _spmm_body,
    out_type=jax.ShapeDtypeStruct((_NC * 2 * _NPAD, _H), jnp.float32),
    mesh=plsc.VectorSubcoreMesh(core_axis_name="c", subcore_axis_name="s"),
    compiler_params=pltpu.CompilerParams(use_tc_tiling_on_sc=False),
    scratch_types=[
        pltpu.VMEM((_CHUNKS, _K), jnp.int32),
        pltpu.VMEM((_CHUNKS, _K), jnp.int32),
        pltpu.VMEM((_CHUNKS, _K), jnp.float32),
        pltpu.VMEM((_K, _H), jnp.float32),
        pltpu.VMEM((_K, _H), jnp.float32),
        pltpu.VMEM((_ROWS_PER_TILE, _H), jnp.float32),
        pltpu.VMEM_SHARED((_NPAD, _H), jnp.float32),
        pltpu.SemaphoreType.DMA,
        pltpu.SemaphoreType.DMA,
    ],
)


_BN = 1000


def _lin1_body(x_ref, wt_ref, b_ref, o_ref):
    o_ref[...] = (jnp.dot(x_ref[...], wt_ref[...],
                          preferred_element_type=jnp.float32) + b_ref[...])


def _lin2_body(p00_ref, p01_ref, p10_ref, p11_ref, wt_ref, b_ref, o_ref):
    hl = jnp.maximum(p00_ref[...] + p10_ref[...], 0.0)
    hr = jnp.maximum(p01_ref[...] + p11_ref[...], 0.0)
    acc = jnp.dot(hl, wt_ref[:_H, :], preferred_element_type=jnp.float32)
    acc += jnp.dot(hr, wt_ref[_H:, :], preferred_element_type=jnp.float32)
    o_ref[...] = acc + b_ref[...]


def _add_body(a0_ref, a1_ref, b0_ref, b1_ref, o_ref):
    o_ref[:, :_H] = a0_ref[...] + b0_ref[...]
    o_ref[:, _H:] = a1_ref[...] + b1_ref[...]


_row_spec = pl.BlockSpec((_BN, _D), lambda i: (i, 0))
_half_spec = pl.BlockSpec((_BN, _H), lambda i: (i, 0))
_mat_spec = pl.BlockSpec((_D, _D), lambda i: (0, 0))
_bias_spec = pl.BlockSpec((1, _D), lambda i: (0, 0))
_out_f32 = jax.ShapeDtypeStruct((_N, _D), jnp.float32)

_linear1 = pl.pallas_call(
    _lin1_body, grid=(_N // _BN,),
    in_specs=[_row_spec, _mat_spec, _bias_spec],
    out_specs=_row_spec, out_shape=_out_f32)

_linear2 = pl.pallas_call(
    _lin2_body, grid=(_N // _BN,),
    in_specs=[_half_spec, _half_spec, _half_spec, _half_spec,
              _mat_spec, _bias_spec],
    out_specs=_row_spec, out_shape=_out_f32)

_addk = pl.pallas_call(
    _add_body, grid=(_N // _BN,),
    in_specs=[_half_spec, _half_spec, _half_spec, _half_spec],
    out_specs=_row_spec, out_shape=_out_f32)


def _partials(p):
    return [p[k * _NPAD:k * _NPAD + _N] for k in range(_NC * 2)]


def kernel(x, edge_index, edge_weight, W1, b1, W2, b2):
    src = edge_index[0].reshape(_E // _K, _K)
    dst = edge_index[1].reshape(_E // _K, _K)
    edge_weight = edge_weight.reshape(_E // _K, _K)
    w1t = W1.T
    w2t = W2.T
    b1r = b1.reshape(1, _D)
    b2r = b2.reshape(1, _D)

    h1 = _linear1(x, w1t, b1r)
    p = _partials(_spmm(h1[:, :_H], h1[:, _H:], src, dst, edge_weight))
    h2 = _linear2(p[0], p[1], p[2], p[3], w2t, b2r)
    q = _partials(_spmm(h2[:, :_H], h2[:, _H:], src, dst, edge_weight))
    return _addk(q[0], q[1], q[2], q[3])


# speedup vs baseline: 2.5667x; 1.4772x over previous
import jax
import jax.numpy as jnp
from jax import lax
from jax.experimental import pallas as pl
from jax.experimental.pallas import tpu as pltpu
from jax.experimental.pallas import tpu_sc as plsc

_N = 10000
_D = 128
_H = _D // 2
_E = 320000
_LANES = 16

_NC = 2
_NS = 16
_EDGES_PER_CORE = _E // _NC
_EDGES_PER_TILE = _E // (_NC * _NS)
_K = 80
_CHUNKS = _EDGES_PER_TILE // _K
_ROWS_PER_TILE = 632
_NPAD = _ROWS_PER_TILE * _NS
_LAST_ROWS = _N - (_NS - 1) * _ROWS_PER_TILE


def _spmm_body(ha_hbm, hb_hbm, src_hbm, dst_hbm, w_hbm, out_hbm,
               src_all, dst_all, w_all, rows0, rows1, rows2, rows3,
               zbuf, accum,
               gs0, gs1, gs2, gs3, ss0, ss1, ss2, ss3):
    bufs = (rows0, rows1, rows2, rows3)
    gsems = (gs0, gs1, gs2, gs3)
    ssems = (ss0, ss1, ss2, ss3)
    cid = lax.axis_index("c")
    sid = lax.axis_index("s")
    row0 = sid * _ROWS_PER_TILE
    chunk0 = cid * (_EDGES_PER_CORE // _K) + sid * _CHUNKS

    pltpu.sync_copy(src_hbm.at[pl.ds(chunk0, _CHUNKS)], src_all)
    pltpu.sync_copy(dst_hbm.at[pl.ds(chunk0, _CHUNKS)], dst_all)
    pltpu.sync_copy(w_hbm.at[pl.ds(chunk0, _CHUNKS)], w_all)

    def _zrow(r, carry):
        for j in range(_H // _LANES):
            zbuf[r, pl.ds(j * _LANES, _LANES)] = jnp.zeros((_LANES,), jnp.float32)
        return carry
    lax.fori_loop(0, _ROWS_PER_TILE, _zrow, 0)

    for half, h_hbm in enumerate((ha_hbm, hb_hbm)):
        @pl.when(sid < _NS - 1)
        def _():
            pltpu.sync_copy(zbuf, accum.at[pl.ds(row0, _ROWS_PER_TILE)])

        @pl.when(sid == _NS - 1)
        def _():
            pltpu.sync_copy(zbuf.at[pl.ds(0, _LAST_ROWS)],
                            accum.at[pl.ds(row0, _LAST_ROWS)])
        plsc.subcore_barrier()

        def _gather(c, buf, sem):
            pltpu.async_copy(h_hbm.at[src_all.at[c]], buf, sem)

        def _wait_g(c, buf, sem):
            pltpu.make_async_copy(h_hbm.at[src_all.at[c]], buf, sem).wait()

        def _scale(c, buf):
            def _grp(g, c2):
                w16 = w_all[c, pl.ds(g * _LANES, _LANES)]
                e0 = g * _LANES
                nb = _H // _LANES
                for i in range(0, _LANES, 2):
                    wva = jnp.full((_LANES,), w16[i], jnp.float32)
                    wvb = jnp.full((_LANES,), w16[i + 1], jnp.float32)
                    va = [buf[e0 + i, pl.ds(j * _LANES, _LANES)]
                          for j in range(nb)]
                    vb = [buf[e0 + i + 1, pl.ds(j * _LANES, _LANES)]
                          for j in range(nb)]
                    for j in range(nb):
                        buf[e0 + i, pl.ds(j * _LANES, _LANES)] = va[j] * wva
                    for j in range(nb):
                        buf[e0 + i + 1, pl.ds(j * _LANES, _LANES)] = vb[j] * wvb
                return c2
            lax.fori_loop(0, _K // _LANES, _grp, 0)

        def _scatter(c, buf, sem):
            pltpu.async_copy(buf, accum.at[dst_all.at[c]], sem, add=True)

        def _wait_s(c, buf, sem):
            pltpu.make_async_copy(buf, accum.at[dst_all.at[c]], sem).wait()

        _gather(0, bufs[0], gsems[0])
        _gather(1, bufs[1], gsems[1])
        _gather(2, bufs[2], gsems[2])

        def _quad(j, carry):
            for l in range(4):
                c = 4 * j + l
                _wait_g(c, bufs[l], gsems[l])
                _scale(c, bufs[l])
                _scatter(c, bufs[l], ssems[l])
                lp = (l - 1) % 4

                @pl.when(c >= 1)
                def _():
                    _wait_s(c - 1, bufs[lp], ssems[lp])
                ln = (l + 3) % 4

                @pl.when(c + 3 < _CHUNKS)
                def _():
                    _gather(c + 3, bufs[ln], gsems[ln])
            return carry
        lax.fori_loop(0, (_CHUNKS - 1) // 4, _quad, 0)
        cz = _CHUNKS - 1
        _wait_g(cz, bufs[0], gsems[0])
        _scale(cz, bufs[0])
        _scatter(cz, bufs[0], ssems[0])
        _wait_s(cz - 1, bufs[3], ssems[3])
        _wait_s(cz, bufs[0], ssems[0])

        plsc.subcore_barrier()
        out0 = (cid * 2 + half) * _NPAD + row0

        @pl.when(sid < _NS - 1)
        def _():
            pltpu.sync_copy(accum.at[pl.ds(row0, _ROWS_PER_TILE)],
                            out_hbm.at[pl.ds(out0, _ROWS_PER_TILE)])

        @pl.when(sid == _NS - 1)
        def _():
            pltpu.sync_copy(accum.at[pl.ds(row0, _LAST_ROWS)],
                            out_hbm.at[pl.ds(out0, _LAST_ROWS)])
        plsc.subcore_barrier()


_spmm = pl.kernel(
    _spmm_body,
    out_type=jax.ShapeDtypeStruct((_NC * 2 * _NPAD, _H), jnp.float32),
    mesh=plsc.VectorSubcoreMesh(core_axis_name="c", subcore_axis_name="s"),
    compiler_params=pltpu.CompilerParams(use_tc_tiling_on_sc=False),
    scratch_types=[
        pltpu.VMEM((_CHUNKS, _K), jnp.int32),
        pltpu.VMEM((_CHUNKS, _K), jnp.int32),
        pltpu.VMEM((_CHUNKS, _K), jnp.float32),
        pltpu.VMEM((_K, _H), jnp.float32),
        pltpu.VMEM((_K, _H), jnp.float32),
        pltpu.VMEM((_K, _H), jnp.float32),
        pltpu.VMEM((_K, _H), jnp.float32),
        pltpu.VMEM((_ROWS_PER_TILE, _H), jnp.float32),
        pltpu.VMEM_SHARED((_N, _H), jnp.float32),
    ] + [pltpu.SemaphoreType.DMA] * 8,
)


_BN = 1000


def _lin1_body(x_ref, wt_ref, b_ref, o_ref):
    o_ref[...] = (jnp.dot(x_ref[...], wt_ref[...],
                          preferred_element_type=jnp.float32) + b_ref[...])


def _lin2_body(p00_ref, p01_ref, p10_ref, p11_ref, wt_ref, b_ref, o_ref):
    hl = jnp.maximum(p00_ref[...] + p10_ref[...], 0.0)
    hr = jnp.maximum(p01_ref[...] + p11_ref[...], 0.0)
    acc = jnp.dot(hl, wt_ref[:_H, :], preferred_element_type=jnp.float32)
    acc += jnp.dot(hr, wt_ref[_H:, :], preferred_element_type=jnp.float32)
    o_ref[...] = acc + b_ref[...]


def _add_body(a0_ref, a1_ref, b0_ref, b1_ref, o_ref):
    o_ref[:, :_H] = a0_ref[...] + b0_ref[...]
    o_ref[:, _H:] = a1_ref[...] + b1_ref[...]


_row_spec = pl.BlockSpec((_BN, _D), lambda i: (i, 0))
_half_spec = pl.BlockSpec((_BN, _H), lambda i: (i, 0))
_mat_spec = pl.BlockSpec((_D, _D), lambda i: (0, 0))
_bias_spec = pl.BlockSpec((1, _D), lambda i: (0, 0))
_out_f32 = jax.ShapeDtypeStruct((_N, _D), jnp.float32)

_linear1 = pl.pallas_call(
    _lin1_body, grid=(_N // _BN,),
    in_specs=[_row_spec, _mat_spec, _bias_spec],
    out_specs=_row_spec, out_shape=_out_f32)

_linear2 = pl.pallas_call(
    _lin2_body, grid=(_N // _BN,),
    in_specs=[_half_spec, _half_spec, _half_spec, _half_spec,
              _mat_spec, _bias_spec],
    out_specs=_row_spec, out_shape=_out_f32)

_addk = pl.pallas_call(
    _add_body, grid=(_N // _BN,),
    in_specs=[_half_spec, _half_spec, _half_spec, _half_spec],
    out_specs=_row_spec, out_shape=_out_f32)


def _partials(p):
    return [p[k * _NPAD:k * _NPAD + _N] for k in range(_NC * 2)]


def kernel(x, edge_index, edge_weight, W1, b1, W2, b2):
    src = edge_index[0].reshape(_E // _K, _K)
    dst = edge_index[1].reshape(_E // _K, _K)
    edge_weight = edge_weight.reshape(_E // _K, _K)
    w1t = W1.T
    w2t = W2.T
    b1r = b1.reshape(1, _D)
    b2r = b2.reshape(1, _D)

    h1 = _linear1(x, w1t, b1r)
    p = _partials(_spmm(h1[:, :_H], h1[:, _H:], src, dst, edge_weight))
    h2 = _linear2(p[0], p[1], p[2], p[3], w2t, b2r)
    q = _partials(_spmm(h2[:, :_H], h2[:, _H:], src, dst, edge_weight))
    return _addk(q[0], q[1], q[2], q[3])


# speedup vs baseline: 2.8467x; 1.1091x over previous
import jax
import jax.numpy as jnp
from jax import lax
from jax.experimental import pallas as pl
from jax.experimental.pallas import tpu as pltpu
from jax.experimental.pallas import tpu_sc as plsc

_N = 10000
_D = 128
_H = _D // 2
_E = 320000
_LANES = 16

_NC = 2
_NS = 16
_EDGES_PER_CORE = _E // _NC
_EDGES_PER_TILE = _E // (_NC * _NS)
_K = 80
_CHUNKS = _EDGES_PER_TILE // _K
_ROWS_PER_TILE = 632
_NPAD = _ROWS_PER_TILE * _NS
_LAST_ROWS = _N - (_NS - 1) * _ROWS_PER_TILE


def _spmm_body(ha_hbm, hb_hbm, src_hbm, dst_hbm, w_hbm,
               o00, o01, o10, o11,
               src_all, dst_all, w_all, rows0, rows1, rows2, rows3,
               zbuf, accum,
               gs0, gs1, gs2, gs3, ss0, ss1, ss2, ss3):
    bufs = (rows0, rows1, rows2, rows3)
    gsems = (gs0, gs1, gs2, gs3)
    ssems = (ss0, ss1, ss2, ss3)
    cid = lax.axis_index("c")
    sid = lax.axis_index("s")
    row0 = sid * _ROWS_PER_TILE
    chunk0 = cid * (_EDGES_PER_CORE // _K) + sid * _CHUNKS

    pltpu.sync_copy(src_hbm.at[pl.ds(chunk0, _CHUNKS)], src_all)
    pltpu.sync_copy(dst_hbm.at[pl.ds(chunk0, _CHUNKS)], dst_all)
    pltpu.sync_copy(w_hbm.at[pl.ds(chunk0, _CHUNKS)], w_all)

    def _zrow(r, carry):
        for j in range(_H // _LANES):
            zbuf[r, pl.ds(j * _LANES, _LANES)] = jnp.zeros((_LANES,), jnp.float32)
        return carry
    lax.fori_loop(0, _ROWS_PER_TILE, _zrow, 0)

    for half, h_hbm in enumerate((ha_hbm, hb_hbm)):
        @pl.when(sid < _NS - 1)
        def _():
            pltpu.sync_copy(zbuf, accum.at[pl.ds(row0, _ROWS_PER_TILE)])

        @pl.when(sid == _NS - 1)
        def _():
            pltpu.sync_copy(zbuf.at[pl.ds(0, _LAST_ROWS)],
                            accum.at[pl.ds(row0, _LAST_ROWS)])
        plsc.subcore_barrier()

        def _gather(c, buf, sem):
            pltpu.async_copy(h_hbm.at[src_all.at[c]], buf, sem)

        def _wait_g(c, buf, sem):
            pltpu.make_async_copy(h_hbm.at[src_all.at[c]], buf, sem).wait()

        def _scale(c, buf):
            def _grp(g, c2):
                w16 = w_all[c, pl.ds(g * _LANES, _LANES)]
                e0 = g * _LANES
                nb = _H // _LANES
                for i in range(0, _LANES, 2):
                    wva = jnp.full((_LANES,), w16[i], jnp.float32)
                    wvb = jnp.full((_LANES,), w16[i + 1], jnp.float32)
                    va = [buf[e0 + i, pl.ds(j * _LANES, _LANES)]
                          for j in range(nb)]
                    vb = [buf[e0 + i + 1, pl.ds(j * _LANES, _LANES)]
                          for j in range(nb)]
                    for j in range(nb):
                        buf[e0 + i, pl.ds(j * _LANES, _LANES)] = va[j] * wva
                    for j in range(nb):
                        buf[e0 + i + 1, pl.ds(j * _LANES, _LANES)] = vb[j] * wvb
                return c2
            lax.fori_loop(0, _K // _LANES, _grp, 0)

        def _scatter(c, buf, sem):
            pltpu.async_copy(buf, accum.at[dst_all.at[c]], sem, add=True)

        def _wait_s(c, buf, sem):
            pltpu.make_async_copy(buf, accum.at[dst_all.at[c]], sem).wait()

        _gather(0, bufs[0], gsems[0])
        _gather(1, bufs[1], gsems[1])
        _gather(2, bufs[2], gsems[2])

        def _quad(j, carry):
            for l in range(4):
                c = 4 * j + l
                _wait_g(c, bufs[l], gsems[l])
                _scale(c, bufs[l])
                _scatter(c, bufs[l], ssems[l])
                lp = (l - 1) % 4

                @pl.when(c >= 1)
                def _():
                    _wait_s(c - 1, bufs[lp], ssems[lp])
                ln = (l + 3) % 4

                @pl.when(c + 3 < _CHUNKS)
                def _():
                    _gather(c + 3, bufs[ln], gsems[ln])
            return carry
        lax.fori_loop(0, (_CHUNKS - 1) // 4, _quad, 0)
        cz = _CHUNKS - 1
        _wait_g(cz, bufs[0], gsems[0])
        _scale(cz, bufs[0])
        _scatter(cz, bufs[0], ssems[0])
        _wait_s(cz - 1, bufs[3], ssems[3])
        _wait_s(cz, bufs[0], ssems[0])

        plsc.subcore_barrier()
        for cval, oref in ((0, (o00, o01)[half]), (1, (o10, o11)[half])):
            @pl.when((cid == cval) & (sid < _NS - 1))
            def _():
                pltpu.sync_copy(accum.at[pl.ds(row0, _ROWS_PER_TILE)],
                                oref.at[pl.ds(row0, _ROWS_PER_TILE)])

            @pl.when((cid == cval) & (sid == _NS - 1))
            def _():
                pltpu.sync_copy(accum.at[pl.ds(row0, _LAST_ROWS)],
                                oref.at[pl.ds(row0, _LAST_ROWS)])
        plsc.subcore_barrier()


_spmm = pl.kernel(
    _spmm_body,
    out_type=[jax.ShapeDtypeStruct((_NPAD, _H), jnp.float32)] * 4,
    mesh=plsc.VectorSubcoreMesh(core_axis_name="c", subcore_axis_name="s"),
    compiler_params=pltpu.CompilerParams(use_tc_tiling_on_sc=False),
    scratch_types=[
        pltpu.VMEM((_CHUNKS, _K), jnp.int32),
        pltpu.VMEM((_CHUNKS, _K), jnp.int32),
        pltpu.VMEM((_CHUNKS, _K), jnp.float32),
        pltpu.VMEM((_K, _H), jnp.float32),
        pltpu.VMEM((_K, _H), jnp.float32),
        pltpu.VMEM((_K, _H), jnp.float32),
        pltpu.VMEM((_K, _H), jnp.float32),
        pltpu.VMEM((_ROWS_PER_TILE, _H), jnp.float32),
        pltpu.VMEM_SHARED((_N, _H), jnp.float32),
    ] + [pltpu.SemaphoreType.DMA] * 8,
)


_BN = 1000


def _lin1_body(x_ref, wt_ref, b_ref, oa_ref, ob_ref):
    acc = (jnp.dot(x_ref[...], wt_ref[...],
                   preferred_element_type=jnp.float32) + b_ref[...])
    oa_ref[...] = acc[:, :_H]
    ob_ref[...] = acc[:, _H:]


def _lin2_body(p00_ref, p01_ref, p10_ref, p11_ref, wt_ref, b_ref,
               oa_ref, ob_ref):
    hl = jnp.maximum(p00_ref[...] + p10_ref[...], 0.0)
    hr = jnp.maximum(p01_ref[...] + p11_ref[...], 0.0)
    acc = jnp.dot(hl, wt_ref[:_H, :], preferred_element_type=jnp.float32)
    acc += jnp.dot(hr, wt_ref[_H:, :], preferred_element_type=jnp.float32)
    acc += b_ref[...]
    oa_ref[...] = acc[:, :_H]
    ob_ref[...] = acc[:, _H:]


def _add_body(a0_ref, a1_ref, b0_ref, b1_ref, o_ref):
    o_ref[:, :_H] = a0_ref[...] + b0_ref[...]
    o_ref[:, _H:] = a1_ref[...] + b1_ref[...]


_row_spec = pl.BlockSpec((_BN, _D), lambda i: (i, 0))
_half_spec = pl.BlockSpec((_BN, _H), lambda i: (i, 0))
_mat_spec = pl.BlockSpec((_D, _D), lambda i: (0, 0))
_bias_spec = pl.BlockSpec((1, _D), lambda i: (0, 0))
_out_f32 = jax.ShapeDtypeStruct((_N, _D), jnp.float32)
_half_f32 = jax.ShapeDtypeStruct((_N, _H), jnp.float32)

_linear1 = pl.pallas_call(
    _lin1_body, grid=(_N // _BN,),
    in_specs=[_row_spec, _mat_spec, _bias_spec],
    out_specs=[_half_spec, _half_spec], out_shape=[_half_f32, _half_f32])

_linear2 = pl.pallas_call(
    _lin2_body, grid=(_N // _BN,),
    in_specs=[_half_spec, _half_spec, _half_spec, _half_spec,
              _mat_spec, _bias_spec],
    out_specs=[_half_spec, _half_spec], out_shape=[_half_f32, _half_f32])

_addk = pl.pallas_call(
    _add_body, grid=(_N // _BN,),
    in_specs=[_half_spec, _half_spec, _half_spec, _half_spec],
    out_specs=_row_spec, out_shape=_out_f32)


def kernel(x, edge_index, edge_weight, W1, b1, W2, b2):
    src = edge_index[0].reshape(_E // _K, _K)
    dst = edge_index[1].reshape(_E // _K, _K)
    edge_weight = edge_weight.reshape(_E // _K, _K)
    w1t = W1.T
    w2t = W2.T
    b1r = b1.reshape(1, _D)
    b2r = b2.reshape(1, _D)

    h1a, h1b = _linear1(x, w1t, b1r)
    p00, p01, p10, p11 = _spmm(h1a, h1b, src, dst, edge_weight)
    h2a, h2b = _linear2(p00, p01, p10, p11, w2t, b2r)
    q00, q01, q10, q11 = _spmm(h2a, h2b, src, dst, edge_weight)
    return _addk(q00, q01, q10, q11)


# speedup vs baseline: 2.9146x; 1.0239x over previous
import jax
import jax.numpy as jnp
from jax import lax
from jax.experimental import pallas as pl
from jax.experimental.pallas import tpu as pltpu
from jax.experimental.pallas import tpu_sc as plsc

_N = 10000
_D = 128
_H = _D // 2
_E = 320000
_LANES = 16

_NC = 2
_NS = 16
_EDGES_PER_CORE = _E // _NC
_EDGES_PER_TILE = _E // (_NC * _NS)
_K = 80
_CHUNKS = _EDGES_PER_TILE // _K
_ROWS_PER_TILE = 632
_NPAD = _ROWS_PER_TILE * _NS
_LAST_ROWS = _N - (_NS - 1) * _ROWS_PER_TILE


def _spmm_body(ha_hbm, hb_hbm, src_hbm, dst_hbm, w_hbm,
               o00, o01, o10, o11,
               src_all, dst_all, w_all, rows0, rows1, rows2, rows3,
               zbuf, accum,
               gs0, gs1, gs2, gs3, ss0, ss1, ss2, ss3):
    bufs = (rows0, rows1, rows2, rows3)
    gsems = (gs0, gs1, gs2, gs3)
    ssems = (ss0, ss1, ss2, ss3)
    cid = lax.axis_index("c")
    sid = lax.axis_index("s")
    row0 = sid * _ROWS_PER_TILE
    chunk0 = cid * (_EDGES_PER_CORE // _K) + sid * _CHUNKS

    pltpu.sync_copy(src_hbm.at[pl.ds(chunk0, _CHUNKS)], src_all)
    pltpu.sync_copy(dst_hbm.at[pl.ds(chunk0, _CHUNKS)], dst_all)
    pltpu.sync_copy(w_hbm.at[pl.ds(chunk0, _CHUNKS)], w_all)

    def _zrow(r, carry):
        for j in range(_H // _LANES):
            zbuf[r, pl.ds(j * _LANES, _LANES)] = jnp.zeros((_LANES,), jnp.float32)
        return carry
    lax.fori_loop(0, _ROWS_PER_TILE, _zrow, 0)

    for half, h_hbm in enumerate((ha_hbm, hb_hbm)):
        @pl.when(sid < _NS - 1)
        def _():
            pltpu.sync_copy(zbuf, accum.at[pl.ds(row0, _ROWS_PER_TILE)])

        @pl.when(sid == _NS - 1)
        def _():
            pltpu.sync_copy(zbuf.at[pl.ds(0, _LAST_ROWS)],
                            accum.at[pl.ds(row0, _LAST_ROWS)])
        plsc.subcore_barrier()

        def _gather(c, buf, sem):
            pltpu.async_copy(h_hbm.at[src_all.at[c]], buf, sem)

        def _wait_g(c, buf, sem):
            pltpu.make_async_copy(h_hbm.at[src_all.at[c]], buf, sem).wait()

        def _scale(c, buf):
            def _grp(g, c2):
                w16 = w_all[c, pl.ds(g * _LANES, _LANES)]
                e0 = g * _LANES
                nb = _H // _LANES
                for i in range(0, _LANES, 4):
                    wvs = [jnp.full((_LANES,), w16[i + a], jnp.float32)
                           for a in range(4)]
                    vs = [[buf[e0 + i + a, pl.ds(j * _LANES, _LANES)]
                           for j in range(nb)] for a in range(4)]
                    for a in range(4):
                        for j in range(nb):
                            buf[e0 + i + a, pl.ds(j * _LANES, _LANES)] = (
                                vs[a][j] * wvs[a])
                return c2
            lax.fori_loop(0, _K // _LANES, _grp, 0)

        def _scatter(c, buf, sem):
            pltpu.async_copy(buf, accum.at[dst_all.at[c]], sem, add=True)

        def _wait_s(c, buf, sem):
            pltpu.make_async_copy(buf, accum.at[dst_all.at[c]], sem).wait()

        _gather(0, bufs[0], gsems[0])
        _gather(1, bufs[1], gsems[1])
        _gather(2, bufs[2], gsems[2])

        def _quad(j, carry):
            for l in range(4):
                c = 4 * j + l
                _wait_g(c, bufs[l], gsems[l])
                _scale(c, bufs[l])
                _scatter(c, bufs[l], ssems[l])
                lp = (l - 1) % 4

                @pl.when(c >= 1)
                def _():
                    _wait_s(c - 1, bufs[lp], ssems[lp])
                ln = (l + 3) % 4

                @pl.when(c + 3 < _CHUNKS)
                def _():
                    _gather(c + 3, bufs[ln], gsems[ln])
            return carry
        lax.fori_loop(0, (_CHUNKS - 1) // 4, _quad, 0)
        cz = _CHUNKS - 1
        _wait_g(cz, bufs[0], gsems[0])
        _scale(cz, bufs[0])
        _scatter(cz, bufs[0], ssems[0])
        _wait_s(cz - 1, bufs[3], ssems[3])
        _wait_s(cz, bufs[0], ssems[0])

        plsc.subcore_barrier()
        for cval, oref in ((0, (o00, o01)[half]), (1, (o10, o11)[half])):
            @pl.when((cid == cval) & (sid < _NS - 1))
            def _():
                pltpu.sync_copy(accum.at[pl.ds(row0, _ROWS_PER_TILE)],
                                oref.at[pl.ds(row0, _ROWS_PER_TILE)])

            @pl.when((cid == cval) & (sid == _NS - 1))
            def _():
                pltpu.sync_copy(accum.at[pl.ds(row0, _LAST_ROWS)],
                                oref.at[pl.ds(row0, _LAST_ROWS)])
        plsc.subcore_barrier()


_spmm = pl.kernel(
    _spmm_body,
    out_type=[jax.ShapeDtypeStruct((_NPAD, _H), jnp.float32)] * 4,
    mesh=plsc.VectorSubcoreMesh(core_axis_name="c", subcore_axis_name="s"),
    compiler_params=pltpu.CompilerParams(use_tc_tiling_on_sc=False),
    scratch_types=[
        pltpu.VMEM((_CHUNKS, _K), jnp.int32),
        pltpu.VMEM((_CHUNKS, _K), jnp.int32),
        pltpu.VMEM((_CHUNKS, _K), jnp.float32),
        pltpu.VMEM((_K, _H), jnp.float32),
        pltpu.VMEM((_K, _H), jnp.float32),
        pltpu.VMEM((_K, _H), jnp.float32),
        pltpu.VMEM((_K, _H), jnp.float32),
        pltpu.VMEM((_ROWS_PER_TILE, _H), jnp.float32),
        pltpu.VMEM_SHARED((_N, _H), jnp.float32),
    ] + [pltpu.SemaphoreType.DMA] * 8,
)


_BN = 1000


def _lin1_body(x_ref, wt_ref, b_ref, oa_ref, ob_ref):
    acc = (jnp.dot(x_ref[...], wt_ref[...],
                   preferred_element_type=jnp.float32) + b_ref[...])
    oa_ref[...] = acc[:, :_H]
    ob_ref[...] = acc[:, _H:]


def _lin2_body(p00_ref, p01_ref, p10_ref, p11_ref, wt_ref, b_ref,
               oa_ref, ob_ref):
    hl = jnp.maximum(p00_ref[...] + p10_ref[...], 0.0)
    hr = jnp.maximum(p01_ref[...] + p11_ref[...], 0.0)
    acc = jnp.dot(hl, wt_ref[:_H, :], preferred_element_type=jnp.float32)
    acc += jnp.dot(hr, wt_ref[_H:, :], preferred_element_type=jnp.float32)
    acc += b_ref[...]
    oa_ref[...] = acc[:, :_H]
    ob_ref[...] = acc[:, _H:]


def _add_body(a0_ref, a1_ref, b0_ref, b1_ref, o_ref):
    o_ref[:, :_H] = a0_ref[...] + b0_ref[...]
    o_ref[:, _H:] = a1_ref[...] + b1_ref[...]


_row_spec = pl.BlockSpec((_BN, _D), lambda i: (i, 0))
_half_spec = pl.BlockSpec((_BN, _H), lambda i: (i, 0))
_mat_spec = pl.BlockSpec((_D, _D), lambda i: (0, 0))
_bias_spec = pl.BlockSpec((1, _D), lambda i: (0, 0))
_out_f32 = jax.ShapeDtypeStruct((_N, _D), jnp.float32)
_half_f32 = jax.ShapeDtypeStruct((_N, _H), jnp.float32)

_linear1 = pl.pallas_call(
    _lin1_body, grid=(_N // _BN,),
    in_specs=[_row_spec, _mat_spec, _bias_spec],
    out_specs=[_half_spec, _half_spec], out_shape=[_half_f32, _half_f32])

_linear2 = pl.pallas_call(
    _lin2_body, grid=(_N // _BN,),
    in_specs=[_half_spec, _half_spec, _half_spec, _half_spec,
              _mat_spec, _bias_spec],
    out_specs=[_half_spec, _half_spec], out_shape=[_half_f32, _half_f32])

_addk = pl.pallas_call(
    _add_body, grid=(_N // _BN,),
    in_specs=[_half_spec, _half_spec, _half_spec, _half_spec],
    out_specs=_row_spec, out_shape=_out_f32)


def kernel(x, edge_index, edge_weight, W1, b1, W2, b2):
    src = edge_index[0].reshape(_E // _K, _K)
    dst = edge_index[1].reshape(_E // _K, _K)
    edge_weight = edge_weight.reshape(_E // _K, _K)
    w1t = W1.T
    w2t = W2.T
    b1r = b1.reshape(1, _D)
    b2r = b2.reshape(1, _D)

    h1a, h1b = _linear1(x, w1t, b1r)
    p00, p01, p10, p11 = _spmm(h1a, h1b, src, dst, edge_weight)
    h2a, h2b = _linear2(p00, p01, p10, p11, w2t, b2r)
    q00, q01, q10, q11 = _spmm(h2a, h2b, src, dst, edge_weight)
    return _addk(q00, q01, q10, q11)


# speedup vs baseline: 3.1286x; 1.0734x over previous
import jax
import jax.numpy as jnp
from jax import lax
from jax.experimental import pallas as pl
from jax.experimental.pallas import tpu as pltpu
from jax.experimental.pallas import tpu_sc as plsc

_N = 10000
_D = 128
_H = _D // 2
_E = 320000
_LANES = 16

_NC = 2
_NS = 16
_EDGES_PER_CORE = _E // _NC
_EDGES_PER_TILE = _E // (_NC * _NS)
_K = 80
_CHUNKS = _EDGES_PER_TILE // _K
_ROWS_PER_TILE = 632
_NPAD = _ROWS_PER_TILE * _NS
_LAST_ROWS = _N - (_NS - 1) * _ROWS_PER_TILE


def _spmm_body(ha_hbm, hb_hbm, src_hbm, dst_hbm, w_hbm,
               o00, o01, o10, o11,
               src_all, dst_all, w_all, rows0, rows1, rows2, rows3,
               frows0, frows1, zbuf, accum,
               gs0, gs1, gs2, gs3, ss0, ss1):
    bufs = (rows0, rows1, rows2, rows3)
    fbufs = (frows0, frows1)
    gsems = (gs0, gs1, gs2, gs3)
    ssems = (ss0, ss1)
    cid = lax.axis_index("c")
    sid = lax.axis_index("s")
    row0 = sid * _ROWS_PER_TILE
    chunk0 = cid * (_EDGES_PER_CORE // _K) + sid * _CHUNKS

    pltpu.sync_copy(src_hbm.at[pl.ds(chunk0, _CHUNKS)], src_all)
    pltpu.sync_copy(dst_hbm.at[pl.ds(chunk0, _CHUNKS)], dst_all)
    pltpu.sync_copy(w_hbm.at[pl.ds(chunk0, _CHUNKS)], w_all)

    def _zrow(r, carry):
        for j in range(_H // _LANES):
            zbuf[r, pl.ds(j * _LANES, _LANES)] = jnp.zeros((_LANES,), jnp.float32)
        return carry
    lax.fori_loop(0, _ROWS_PER_TILE, _zrow, 0)

    for half, h_hbm in enumerate((ha_hbm, hb_hbm)):
        @pl.when(sid < _NS - 1)
        def _():
            pltpu.sync_copy(zbuf, accum.at[pl.ds(row0, _ROWS_PER_TILE)])

        @pl.when(sid == _NS - 1)
        def _():
            pltpu.sync_copy(zbuf.at[pl.ds(0, _LAST_ROWS)],
                            accum.at[pl.ds(row0, _LAST_ROWS)])
        plsc.subcore_barrier()

        def _gather(c, buf, sem):
            pltpu.async_copy(h_hbm.at[src_all.at[c]], buf, sem)

        def _wait_g(c, buf, sem):
            pltpu.make_async_copy(h_hbm.at[src_all.at[c]], buf, sem).wait()

        def _scale(c, buf, fbuf):
            def _grp(g, c2):
                w16 = w_all[c, pl.ds(g * _LANES, _LANES)]
                e0 = g * _LANES
                ng = _H // 32
                for i in range(0, _LANES, 4):
                    wvs = [jnp.full((_LANES,), w16[i + a], jnp.float32)
                           for a in range(4)]
                    xs = [[buf[e0 + i + a, pl.ds(j * 32, 32)]
                           for j in range(ng)] for a in range(4)]
                    for a in range(4):
                        for j in range(ng):
                            lo, hi = plsc.unpack(
                                xs[a][j], format=plsc.PackFormat.INTERLEAVED)
                            fbuf[e0 + i + a, pl.ds(j * 32, _LANES)] = (
                                lo * wvs[a])
                            fbuf[e0 + i + a, pl.ds(j * 32 + _LANES, _LANES)] = (
                                hi * wvs[a])
                return c2
            lax.fori_loop(0, _K // _LANES, _grp, 0)

        def _scatter(c, buf, sem):
            pltpu.async_copy(buf, accum.at[dst_all.at[c]], sem, add=True)

        def _wait_s(c, buf, sem):
            pltpu.make_async_copy(buf, accum.at[dst_all.at[c]], sem).wait()

        _gather(0, bufs[0], gsems[0])
        _gather(1, bufs[1], gsems[1])
        _gather(2, bufs[2], gsems[2])

        def _quad(j, carry):
            for l in range(4):
                c = 4 * j + l
                _wait_g(c, bufs[l], gsems[l])
                lf = l % 2

                @pl.when(c >= 2)
                def _():
                    _wait_s(c - 2, fbufs[lf], ssems[lf])
                _scale(c, bufs[l], fbufs[lf])
                _scatter(c, fbufs[lf], ssems[lf])
                ln = (l + 3) % 4

                @pl.when(c + 3 < _CHUNKS)
                def _():
                    _gather(c + 3, bufs[ln], gsems[ln])
            return carry
        lax.fori_loop(0, (_CHUNKS - 1) // 4, _quad, 0)
        cz = _CHUNKS - 1
        _wait_g(cz, bufs[0], gsems[0])
        _wait_s(cz - 2, fbufs[0], ssems[0])
        _scale(cz, bufs[0], fbufs[0])
        _scatter(cz, fbufs[0], ssems[0])
        _wait_s(cz - 1, fbufs[1], ssems[1])
        _wait_s(cz, fbufs[0], ssems[0])

        plsc.subcore_barrier()
        for cval, oref in ((0, (o00, o01)[half]), (1, (o10, o11)[half])):
            @pl.when((cid == cval) & (sid < _NS - 1))
            def _():
                pltpu.sync_copy(accum.at[pl.ds(row0, _ROWS_PER_TILE)],
                                oref.at[pl.ds(row0, _ROWS_PER_TILE)])

            @pl.when((cid == cval) & (sid == _NS - 1))
            def _():
                pltpu.sync_copy(accum.at[pl.ds(row0, _LAST_ROWS)],
                                oref.at[pl.ds(row0, _LAST_ROWS)])
        plsc.subcore_barrier()


_spmm = pl.kernel(
    _spmm_body,
    out_type=[jax.ShapeDtypeStruct((_NPAD, _H), jnp.float32)] * 4,
    mesh=plsc.VectorSubcoreMesh(core_axis_name="c", subcore_axis_name="s"),
    compiler_params=pltpu.CompilerParams(use_tc_tiling_on_sc=False,
                                         needs_layout_passes=False),
    scratch_types=[
        pltpu.VMEM((_CHUNKS, _K), jnp.int32),
        pltpu.VMEM((_CHUNKS, _K), jnp.int32),
        pltpu.VMEM((_CHUNKS, _K), jnp.float32),
        pltpu.VMEM((_K, _H), jnp.bfloat16),
        pltpu.VMEM((_K, _H), jnp.bfloat16),
        pltpu.VMEM((_K, _H), jnp.bfloat16),
        pltpu.VMEM((_K, _H), jnp.bfloat16),
        pltpu.VMEM((_K, _H), jnp.float32),
        pltpu.VMEM((_K, _H), jnp.float32),
        pltpu.VMEM((_ROWS_PER_TILE, _H), jnp.float32),
        pltpu.VMEM_SHARED((_N, _H), jnp.float32),
    ] + [pltpu.SemaphoreType.DMA] * 6,
)


_BN = 1000


def _lin1_body(x_ref, wt_ref, b_ref, oa_ref, ob_ref):
    acc = (jnp.dot(x_ref[...], wt_ref[...],
                   preferred_element_type=jnp.float32) + b_ref[...])
    oa_ref[...] = acc[:, :_H].astype(jnp.bfloat16)
    ob_ref[...] = acc[:, _H:].astype(jnp.bfloat16)


def _lin2_body(p00_ref, p01_ref, p10_ref, p11_ref, wt_ref, b_ref,
               oa_ref, ob_ref):
    hl = jnp.maximum(p00_ref[...] + p10_ref[...], 0.0)
    hr = jnp.maximum(p01_ref[...] + p11_ref[...], 0.0)
    acc = jnp.dot(hl, wt_ref[:_H, :], preferred_element_type=jnp.float32)
    acc += jnp.dot(hr, wt_ref[_H:, :], preferred_element_type=jnp.float32)
    acc += b_ref[...]
    oa_ref[...] = acc[:, :_H].astype(jnp.bfloat16)
    ob_ref[...] = acc[:, _H:].astype(jnp.bfloat16)


def _add_body(a0_ref, a1_ref, b0_ref, b1_ref, o_ref):
    o_ref[:, :_H] = a0_ref[...] + b0_ref[...]
    o_ref[:, _H:] = a1_ref[...] + b1_ref[...]


_row_spec = pl.BlockSpec((_BN, _D), lambda i: (i, 0))
_half_spec = pl.BlockSpec((_BN, _H), lambda i: (i, 0))
_mat_spec = pl.BlockSpec((_D, _D), lambda i: (0, 0))
_bias_spec = pl.BlockSpec((1, _D), lambda i: (0, 0))
_out_f32 = jax.ShapeDtypeStruct((_N, _D), jnp.float32)
_half_bf16 = jax.ShapeDtypeStruct((_N, _H), jnp.bfloat16)

_linear1 = pl.pallas_call(
    _lin1_body, grid=(_N // _BN,),
    in_specs=[_row_spec, _mat_spec, _bias_spec],
    out_specs=[_half_spec, _half_spec], out_shape=[_half_bf16, _half_bf16])

_linear2 = pl.pallas_call(
    _lin2_body, grid=(_N // _BN,),
    in_specs=[_half_spec, _half_spec, _half_spec, _half_spec,
              _mat_spec, _bias_spec],
    out_specs=[_half_spec, _half_spec], out_shape=[_half_bf16, _half_bf16])

_addk = pl.pallas_call(
    _add_body, grid=(_N // _BN,),
    in_specs=[_half_spec, _half_spec, _half_spec, _half_spec],
    out_specs=_row_spec, out_shape=_out_f32)


def kernel(x, edge_index, edge_weight, W1, b1, W2, b2):
    src = edge_index[0].reshape(_E // _K, _K)
    dst = edge_index[1].reshape(_E // _K, _K)
    edge_weight = edge_weight.reshape(_E // _K, _K)
    perm = jnp.arange(_D).reshape(_D // 32, 2, _LANES)
    perm = perm.transpose(0, 2, 1).reshape(-1)
    w1t = W1.T[:, perm]
    w2t = W2.T[:, perm]
    b1r = b1[perm].reshape(1, _D)
    b2r = b2[perm].reshape(1, _D)

    h1a, h1b = _linear1(x, w1t, b1r)
    p00, p01, p10, p11 = _spmm(h1a, h1b, src, dst, edge_weight)
    h2a, h2b = _linear2(p00, p01, p10, p11, w2t, b2r)
    q00, q01, q10, q11 = _spmm(h2a, h2b, src, dst, edge_weight)
    return _addk(q00, q01, q10, q11)


# speedup vs baseline: 3.1298x; 1.0004x over previous
import jax
import jax.numpy as jnp
from jax import lax
from jax.experimental import pallas as pl
from jax.experimental.pallas import tpu as pltpu
from jax.experimental.pallas import tpu_sc as plsc

_N = 10000
_D = 128
_H = _D // 2
_E = 320000
_LANES = 16

_NC = 2
_NS = 16
_EDGES_PER_CORE = _E // _NC
_EDGES_PER_TILE = _E // (_NC * _NS)
_K = 80
_CHUNKS = _EDGES_PER_TILE // _K
_ROWS_PER_TILE = 632
_NPAD = _ROWS_PER_TILE * _NS
_LAST_ROWS = _N - (_NS - 1) * _ROWS_PER_TILE


def _spmm_body(ha_hbm, hb_hbm, src_hbm, dst_hbm, w_hbm,
               o00, o01, o10, o11,
               src_all, dst_all, w_all, rows0, rows1, rows2, rows3,
               frows0, frows1, zbuf, accum,
               gs0, gs1, gs2, gs3, ss0, ss1):
    bufs = (rows0, rows1, rows2, rows3)
    fbufs = (frows0, frows1)
    gsems = (gs0, gs1, gs2, gs3)
    ssems = (ss0, ss1)
    cid = lax.axis_index("c")
    sid = lax.axis_index("s")
    row0 = sid * _ROWS_PER_TILE
    chunk0 = cid * (_EDGES_PER_CORE // _K) + sid * _CHUNKS

    pltpu.sync_copy(src_hbm.at[pl.ds(chunk0, _CHUNKS)], src_all)
    pltpu.sync_copy(dst_hbm.at[pl.ds(chunk0, _CHUNKS)], dst_all)
    pltpu.sync_copy(w_hbm.at[pl.ds(chunk0, _CHUNKS)], w_all)

    def _zrow(r, carry):
        for j in range(_H // _LANES):
            zbuf[r, pl.ds(j * _LANES, _LANES)] = jnp.zeros((_LANES,), jnp.float32)
        return carry
    lax.fori_loop(0, _ROWS_PER_TILE, _zrow, 0)

    for half, h_hbm in enumerate((ha_hbm, hb_hbm)):
        @pl.when(sid < _NS - 1)
        def _():
            pltpu.sync_copy(zbuf, accum.at[pl.ds(row0, _ROWS_PER_TILE)])

        @pl.when(sid == _NS - 1)
        def _():
            pltpu.sync_copy(zbuf.at[pl.ds(0, _LAST_ROWS)],
                            accum.at[pl.ds(row0, _LAST_ROWS)])
        plsc.subcore_barrier()

        def _gather(c, buf, sem):
            pltpu.async_copy(h_hbm.at[src_all.at[c]], buf, sem)

        def _wait_g(c, buf, sem):
            pltpu.make_async_copy(h_hbm.at[src_all.at[c]], buf, sem).wait()

        def _scale(c, buf, fbuf):
            def _grp(g, c2):
                w16 = w_all[c, pl.ds(g * _LANES, _LANES)]
                e0 = g * _LANES
                ng = _H // 32
                for i in range(0, _LANES, 4):
                    wvs = [jnp.full((_LANES,), w16[i + a], jnp.float32)
                           for a in range(4)]
                    xs = [[buf[e0 + i + a, pl.ds(j * 32, 32)]
                           for j in range(ng)] for a in range(4)]
                    for a in range(4):
                        for j in range(ng):
                            lo, hi = plsc.unpack(
                                xs[a][j], format=plsc.PackFormat.INTERLEAVED)
                            fbuf[e0 + i + a, pl.ds(j * 32, _LANES)] = (
                                lo * wvs[a])
                            fbuf[e0 + i + a, pl.ds(j * 32 + _LANES, _LANES)] = (
                                hi * wvs[a])
                return c2
            lax.fori_loop(0, _K // _LANES, _grp, 0)

        def _scatter(c, buf, sem):
            pltpu.async_copy(buf, accum.at[dst_all.at[c]], sem, add=True)

        def _wait_s(c, buf, sem):
            pltpu.make_async_copy(buf, accum.at[dst_all.at[c]], sem).wait()

        _gather(0, bufs[0], gsems[0])
        _gather(1, bufs[1], gsems[1])
        _gather(2, bufs[2], gsems[2])

        def _quad(j, carry):
            for l in range(4):
                c = 4 * j + l
                _wait_g(c, bufs[l], gsems[l])
                lf = l % 2

                @pl.when(c >= 2)
                def _():
                    _wait_s(c - 2, fbufs[lf], ssems[lf])
                _scale(c, bufs[l], fbufs[lf])
                _scatter(c, fbufs[lf], ssems[lf])
                ln = (l + 3) % 4

                @pl.when(c + 3 < _CHUNKS)
                def _():
                    _gather(c + 3, bufs[ln], gsems[ln])
            return carry
        lax.fori_loop(0, (_CHUNKS - 1) // 4, _quad, 0)
        cz = _CHUNKS - 1
        _wait_g(cz, bufs[0], gsems[0])
        _wait_s(cz - 2, fbufs[0], ssems[0])
        _scale(cz, bufs[0], fbufs[0])
        _scatter(cz, fbufs[0], ssems[0])
        _wait_s(cz - 1, fbufs[1], ssems[1])
        _wait_s(cz, fbufs[0], ssems[0])

        plsc.subcore_barrier()
        for cval, oref in ((0, (o00, o01)[half]), (1, (o10, o11)[half])):
            @pl.when((cid == cval) & (sid < _NS - 1))
            def _():
                pltpu.sync_copy(accum.at[pl.ds(row0, _ROWS_PER_TILE)],
                                oref.at[pl.ds(row0, _ROWS_PER_TILE)])

            @pl.when((cid == cval) & (sid == _NS - 1))
            def _():
                pltpu.sync_copy(accum.at[pl.ds(row0, _LAST_ROWS)],
                                oref.at[pl.ds(row0, _LAST_ROWS)])


_spmm = pl.kernel(
    _spmm_body,
    out_type=[jax.ShapeDtypeStruct((_NPAD, _H), jnp.float32)] * 4,
    mesh=plsc.VectorSubcoreMesh(core_axis_name="c", subcore_axis_name="s"),
    compiler_params=pltpu.CompilerParams(use_tc_tiling_on_sc=False,
                                         needs_layout_passes=False),
    scratch_types=[
        pltpu.VMEM((_CHUNKS, _K), jnp.int32),
        pltpu.VMEM((_CHUNKS, _K), jnp.int32),
        pltpu.VMEM((_CHUNKS, _K), jnp.float32),
        pltpu.VMEM((_K, _H), jnp.bfloat16),
        pltpu.VMEM((_K, _H), jnp.bfloat16),
        pltpu.VMEM((_K, _H), jnp.bfloat16),
        pltpu.VMEM((_K, _H), jnp.bfloat16),
        pltpu.VMEM((_K, _H), jnp.float32),
        pltpu.VMEM((_K, _H), jnp.float32),
        pltpu.VMEM((_ROWS_PER_TILE, _H), jnp.float32),
        pltpu.VMEM_SHARED((_N, _H), jnp.float32),
    ] + [pltpu.SemaphoreType.DMA] * 6,
)


_BN = 1000


def _lin1_body(x_ref, wt_ref, b_ref, oa_ref, ob_ref):
    acc = (jnp.dot(x_ref[...], wt_ref[...],
                   preferred_element_type=jnp.float32) + b_ref[...])
    oa_ref[...] = acc[:, :_H].astype(jnp.bfloat16)
    ob_ref[...] = acc[:, _H:].astype(jnp.bfloat16)


def _lin2_body(p00_ref, p01_ref, p10_ref, p11_ref, wt_ref, b_ref,
               oa_ref, ob_ref):
    hl = jnp.maximum(p00_ref[...] + p10_ref[...], 0.0)
    hr = jnp.maximum(p01_ref[...] + p11_ref[...], 0.0)
    acc = jnp.dot(hl, wt_ref[:_H, :], preferred_element_type=jnp.float32)
    acc += jnp.dot(hr, wt_ref[_H:, :], preferred_element_type=jnp.float32)
    acc += b_ref[...]
    oa_ref[...] = acc[:, :_H].astype(jnp.bfloat16)
    ob_ref[...] = acc[:, _H:].astype(jnp.bfloat16)


def _add_body(a0_ref, a1_ref, b0_ref, b1_ref, o_ref):
    o_ref[:, :_H] = a0_ref[...] + b0_ref[...]
    o_ref[:, _H:] = a1_ref[...] + b1_ref[...]


_row_spec = pl.BlockSpec((_BN, _D), lambda i: (i, 0))
_half_spec = pl.BlockSpec((_BN, _H), lambda i: (i, 0))
_mat_spec = pl.BlockSpec((_D, _D), lambda i: (0, 0))
_bias_spec = pl.BlockSpec((1, _D), lambda i: (0, 0))
_out_f32 = jax.ShapeDtypeStruct((_N, _D), jnp.float32)
_half_bf16 = jax.ShapeDtypeStruct((_N, _H), jnp.bfloat16)

_linear1 = pl.pallas_call(
    _lin1_body, grid=(_N // _BN,),
    in_specs=[_row_spec, _mat_spec, _bias_spec],
    out_specs=[_half_spec, _half_spec], out_shape=[_half_bf16, _half_bf16])

_linear2 = pl.pallas_call(
    _lin2_body, grid=(_N // _BN,),
    in_specs=[_half_spec, _half_spec, _half_spec, _half_spec,
              _mat_spec, _bias_spec],
    out_specs=[_half_spec, _half_spec], out_shape=[_half_bf16, _half_bf16])

_addk = pl.pallas_call(
    _add_body, grid=(_N // _BN,),
    in_specs=[_half_spec, _half_spec, _half_spec, _half_spec],
    out_specs=_row_spec, out_shape=_out_f32)


def kernel(x, edge_index, edge_weight, W1, b1, W2, b2):
    src = edge_index[0].reshape(_E // _K, _K)
    dst = edge_index[1].reshape(_E // _K, _K)
    edge_weight = edge_weight.reshape(_E // _K, _K)
    perm = jnp.arange(_D).reshape(_D // 32, 2, _LANES)
    perm = perm.transpose(0, 2, 1).reshape(-1)
    w1t = W1.T[:, perm]
    w2t = W2.T[:, perm]
    b1r = b1[perm].reshape(1, _D)
    b2r = b2[perm].reshape(1, _D)

    h1a, h1b = _linear1(x, w1t, b1r)
    p00, p01, p10, p11 = _spmm(h1a, h1b, src, dst, edge_weight)
    h2a, h2b = _linear2(p00, p01, p10, p11, w2t, b2r)
    q00, q01, q10, q11 = _spmm(h2a, h2b, src, dst, edge_weight)
    return _addk(q00, q01, q10, q11)


# speedup vs baseline: 3.1919x; 1.0198x over previous
import jax
import jax.numpy as jnp
from jax import lax
from jax.experimental import pallas as pl
from jax.experimental.pallas import tpu as pltpu
from jax.experimental.pallas import tpu_sc as plsc

_N = 10000
_D = 128
_H = _D // 2
_E = 320000
_LANES = 16

_NC = 2
_NS = 16
_EDGES_PER_CORE = _E // _NC
_EDGES_PER_TILE = _E // (_NC * _NS)
_K = 80
_CHUNKS = _EDGES_PER_TILE // _K
_ROWS_PER_TILE = 632
_NPAD = _ROWS_PER_TILE * _NS
_LAST_ROWS = _N - (_NS - 1) * _ROWS_PER_TILE


def _spmm_body(ha_hbm, hb_hbm, src_hbm, dst_hbm, w_hbm,
               o00, o01, o10, o11,
               src_all, dst_all, w_all, rows0, rows1, rows2, rows3,
               frows0, frows1, zbuf, accum,
               gs0, gs1, gs2, gs3, ss0, ss1):
    bufs = (rows0, rows1, rows2, rows3)
    fbufs = (frows0, frows1)
    gsems = (gs0, gs1, gs2, gs3)
    ssems = (ss0, ss1)
    cid = lax.axis_index("c")
    sid = lax.axis_index("s")
    row0 = sid * _ROWS_PER_TILE
    chunk0 = cid * (_EDGES_PER_CORE // _K) + sid * _CHUNKS

    e0 = pltpu.async_copy(src_hbm.at[pl.ds(chunk0, _CHUNKS)], src_all, gs0)
    e1 = pltpu.async_copy(dst_hbm.at[pl.ds(chunk0, _CHUNKS)], dst_all, gs1)
    e2 = pltpu.async_copy(w_hbm.at[pl.ds(chunk0, _CHUNKS)], w_all, gs2)

    def _zrow(r, carry):
        for j in range(_H // _LANES):
            zbuf[r, pl.ds(j * _LANES, _LANES)] = jnp.zeros((_LANES,), jnp.float32)
        return carry
    lax.fori_loop(0, _ROWS_PER_TILE, _zrow, 0)
    e0.wait()
    e1.wait()
    e2.wait()

    for half, h_hbm in enumerate((ha_hbm, hb_hbm)):
        @pl.when(sid < _NS - 1)
        def _():
            pltpu.sync_copy(zbuf, accum.at[pl.ds(row0, _ROWS_PER_TILE)])

        @pl.when(sid == _NS - 1)
        def _():
            pltpu.sync_copy(zbuf.at[pl.ds(0, _LAST_ROWS)],
                            accum.at[pl.ds(row0, _LAST_ROWS)])
        plsc.subcore_barrier()

        def _gather(c, buf, sem):
            pltpu.async_copy(h_hbm.at[src_all.at[c]], buf, sem)

        def _wait_g(c, buf, sem):
            pltpu.make_async_copy(h_hbm.at[src_all.at[c]], buf, sem).wait()

        def _scale(c, buf, fbuf):
            def _grp(g, c2):
                w16 = w_all[c, pl.ds(g * _LANES, _LANES)]
                e0 = g * _LANES
                ng = _H // 32
                for i in range(0, _LANES, 4):
                    wvs = [jnp.full((_LANES,), w16[i + a], jnp.float32)
                           for a in range(4)]
                    xs = [[buf[e0 + i + a, pl.ds(j * 32, 32)]
                           for j in range(ng)] for a in range(4)]
                    for a in range(4):
                        for j in range(ng):
                            lo, hi = plsc.unpack(
                                xs[a][j], format=plsc.PackFormat.INTERLEAVED)
                            fbuf[e0 + i + a, pl.ds(j * 32, _LANES)] = (
                                lo * wvs[a])
                            fbuf[e0 + i + a, pl.ds(j * 32 + _LANES, _LANES)] = (
                                hi * wvs[a])
                return c2
            lax.fori_loop(0, _K // _LANES, _grp, 0)

        def _scatter(c, buf, sem):
            pltpu.async_copy(buf, accum.at[dst_all.at[c]], sem, add=True)

        def _wait_s(c, buf, sem):
            pltpu.make_async_copy(buf, accum.at[dst_all.at[c]], sem).wait()

        _gather(0, bufs[0], gsems[0])
        _gather(1, bufs[1], gsems[1])
        _gather(2, bufs[2], gsems[2])

        def _quad(j, carry):
            for l in range(4):
                c = 4 * j + l
                _wait_g(c, bufs[l], gsems[l])
                lf = l % 2

                @pl.when(c >= 2)
                def _():
                    _wait_s(c - 2, fbufs[lf], ssems[lf])
                _scale(c, bufs[l], fbufs[lf])
                _scatter(c, fbufs[lf], ssems[lf])
                ln = (l + 3) % 4

                @pl.when(c + 3 < _CHUNKS)
                def _():
                    _gather(c + 3, bufs[ln], gsems[ln])
            return carry
        lax.fori_loop(0, (_CHUNKS - 1) // 4, _quad, 0)
        cz = _CHUNKS - 1
        _wait_g(cz, bufs[0], gsems[0])
        _wait_s(cz - 2, fbufs[0], ssems[0])
        _scale(cz, bufs[0], fbufs[0])
        _scatter(cz, fbufs[0], ssems[0])
        _wait_s(cz - 1, fbufs[1], ssems[1])
        _wait_s(cz, fbufs[0], ssems[0])

        plsc.subcore_barrier()
        for cval, oref in ((0, (o00, o01)[half]), (1, (o10, o11)[half])):
            @pl.when((cid == cval) & (sid < _NS - 1))
            def _():
                pltpu.sync_copy(accum.at[pl.ds(row0, _ROWS_PER_TILE)],
                                oref.at[pl.ds(row0, _ROWS_PER_TILE)])

            @pl.when((cid == cval) & (sid == _NS - 1))
            def _():
                pltpu.sync_copy(accum.at[pl.ds(row0, _LAST_ROWS)],
                                oref.at[pl.ds(row0, _LAST_ROWS)])


_spmm = pl.kernel(
    _spmm_body,
    out_type=[jax.ShapeDtypeStruct((_NPAD, _H), jnp.float32)] * 4,
    mesh=plsc.VectorSubcoreMesh(core_axis_name="c", subcore_axis_name="s"),
    compiler_params=pltpu.CompilerParams(use_tc_tiling_on_sc=False,
                                         needs_layout_passes=False),
    scratch_types=[
        pltpu.VMEM((_CHUNKS, _K), jnp.int32),
        pltpu.VMEM((_CHUNKS, _K), jnp.int32),
        pltpu.VMEM((_CHUNKS, _K), jnp.float32),
        pltpu.VMEM((_K, _H), jnp.bfloat16),
        pltpu.VMEM((_K, _H), jnp.bfloat16),
        pltpu.VMEM((_K, _H), jnp.bfloat16),
        pltpu.VMEM((_K, _H), jnp.bfloat16),
        pltpu.VMEM((_K, _H), jnp.float32),
        pltpu.VMEM((_K, _H), jnp.float32),
        pltpu.VMEM((_ROWS_PER_TILE, _H), jnp.float32),
        pltpu.VMEM_SHARED((_N, _H), jnp.float32),
    ] + [pltpu.SemaphoreType.DMA] * 6,
)


_BN = 1000


def _lin1_body(x_ref, wt_ref, b_ref, oa_ref, ob_ref):
    acc = (jnp.dot(x_ref[...], wt_ref[...],
                   preferred_element_type=jnp.float32) + b_ref[...])
    oa_ref[...] = acc[:, :_H].astype(jnp.bfloat16)
    ob_ref[...] = acc[:, _H:].astype(jnp.bfloat16)


def _lin2_body(p00_ref, p01_ref, p10_ref, p11_ref, wt_ref, b_ref,
               oa_ref, ob_ref):
    hl = jnp.maximum(p00_ref[...] + p10_ref[...], 0.0)
    hr = jnp.maximum(p01_ref[...] + p11_ref[...], 0.0)
    acc = jnp.dot(hl, wt_ref[:_H, :], preferred_element_type=jnp.float32)
    acc += jnp.dot(hr, wt_ref[_H:, :], preferred_element_type=jnp.float32)
    acc += b_ref[...]
    oa_ref[...] = acc[:, :_H].astype(jnp.bfloat16)
    ob_ref[...] = acc[:, _H:].astype(jnp.bfloat16)


def _add_body(a0_ref, a1_ref, b0_ref, b1_ref, o_ref):
    o_ref[:, :_H] = a0_ref[...] + b0_ref[...]
    o_ref[:, _H:] = a1_ref[...] + b1_ref[...]


_row_spec = pl.BlockSpec((_BN, _D), lambda i: (i, 0))
_half_spec = pl.BlockSpec((_BN, _H), lambda i: (i, 0))
_mat_spec = pl.BlockSpec((_D, _D), lambda i: (0, 0))
_bias_spec = pl.BlockSpec((1, _D), lambda i: (0, 0))
_out_f32 = jax.ShapeDtypeStruct((_N, _D), jnp.float32)
_half_bf16 = jax.ShapeDtypeStruct((_N, _H), jnp.bfloat16)

_linear1 = pl.pallas_call(
    _lin1_body, grid=(_N // _BN,),
    in_specs=[_row_spec, _mat_spec, _bias_spec],
    out_specs=[_half_spec, _half_spec], out_shape=[_half_bf16, _half_bf16])

_linear2 = pl.pallas_call(
    _lin2_body, grid=(_N // _BN,),
    in_specs=[_half_spec, _half_spec, _half_spec, _half_spec,
              _mat_spec, _bias_spec],
    out_specs=[_half_spec, _half_spec], out_shape=[_half_bf16, _half_bf16])

_addk = pl.pallas_call(
    _add_body, grid=(_N // _BN,),
    in_specs=[_half_spec, _half_spec, _half_spec, _half_spec],
    out_specs=_row_spec, out_shape=_out_f32)


def kernel(x, edge_index, edge_weight, W1, b1, W2, b2):
    src = edge_index[0].reshape(_E // _K, _K)
    dst = edge_index[1].reshape(_E // _K, _K)
    edge_weight = edge_weight.reshape(_E // _K, _K)
    perm = jnp.arange(_D).reshape(_D // 32, 2, _LANES)
    perm = perm.transpose(0, 2, 1).reshape(-1)
    w1t = W1.T[:, perm]
    w2t = W2.T[:, perm]
    b1r = b1[perm].reshape(1, _D)
    b2r = b2[perm].reshape(1, _D)

    h1a, h1b = _linear1(x, w1t, b1r)
    p00, p01, p10, p11 = _spmm(h1a, h1b, src, dst, edge_weight)
    h2a, h2b = _linear2(p00, p01, p10, p11, w2t, b2r)
    q00, q01, q10, q11 = _spmm(h2a, h2b, src, dst, edge_weight)
    return _addk(q00, q01, q10, q11)
